# Initial kernel scaffold; baseline (speedup 1.0000x reference)
import jax
import jax.numpy as jnp
from jax.experimental import pallas as pl


def kernel(x, edge_index, batch, W1, b1, W2, b2):
    raise NotImplementedError("write your pallas kernel here")


# speedup vs baseline: 18.4714x; 18.4714x over previous
import dataclasses

import jax
import jax.numpy as jnp
from jax import lax
from jax.experimental import pallas as pl
from jax.experimental.pallas import tpu as pltpu
from jax.experimental.pallas import tpu_sc as plsc

N, E, D, H, G = 10000, 320000, 128, 128, 64
NC, NS = 2, 16
NW = NC * NS
EPW = E // NW
WIN = 80
NWIN = EPW // WIN
NPAD = 10240
RPS = NPAD // NS
DCH = NPAD // NS
F32 = jnp.float32

_MESH = plsc.VectorSubcoreMesh(core_axis_name="c", subcore_axis_name="s")

_SC_PARAMS = pltpu.CompilerParams()
if "needs_layout_passes" in pltpu.CompilerParams.__dataclass_fields__:
    _SC_PARAMS = dataclasses.replace(_SC_PARAMS, needs_layout_passes=False)


def _sc_degree_body(dst_hbm, deg0_hbm, deg1_hbm, didx, dpart, tmp, accd,
                    parts, sem):
    cid = lax.axis_index("c")
    sid = lax.axis_index("s")
    gw = cid * NS + sid

    @pl.loop(0, NPAD, step=16)
    def _(i):
        dpart[pl.ds(i, 16)] = jnp.zeros((16,), F32)

    pltpu.async_copy(dst_hbm.at[pl.ds(gw * EPW, EPW)], didx, sem).wait()
    ones = jnp.full((16,), 1.0, F32)

    @pl.loop(0, EPW, step=16)
    def _(i):
        plsc.addupdate_scatter(dpart, [didx[pl.ds(i, 16)]], ones)

    pltpu.sync_copy(dpart, parts.at[pl.ds(pl.multiple_of(sid * NPAD, 8),
                                          NPAD)])
    plsc.subcore_barrier()

    @pl.loop(0, DCH, step=16)
    def _(i):
        accd[pl.ds(i, 16)] = jnp.zeros((16,), F32)

    for r in range(NS):
        off = pl.multiple_of(r * NPAD + sid * DCH, 8)
        pltpu.sync_copy(parts.at[pl.ds(off, DCH)], tmp)

        @pl.loop(0, DCH, step=16)
        def _(i):
            accd[pl.ds(i, 16)] = accd[pl.ds(i, 16)] + tmp[pl.ds(i, 16)]

    doff = pl.multiple_of(sid * DCH, 8)

    @pl.when(cid == 0)
    def _():
        pltpu.sync_copy(accd, deg0_hbm.at[pl.ds(doff, DCH)])

    @pl.when(cid == 1)
    def _():
        pltpu.sync_copy(accd, deg1_hbm.at[pl.ds(doff, DCH)])


_sc_degree = pl.kernel(
    _sc_degree_body,
    out_type=(jax.ShapeDtypeStruct((NPAD,), F32),
              jax.ShapeDtypeStruct((NPAD,), F32)),
    mesh=_MESH,
    scratch_types=[
        pltpu.VMEM((EPW,), jnp.int32),
        pltpu.VMEM((NPAD,), F32),
        pltpu.VMEM((DCH,), F32),
        pltpu.VMEM((DCH,), F32),
        pltpu.VMEM_SHARED((NS * NPAD,), F32),
        pltpu.SemaphoreType.DMA,
    ],
    compiler_params=_SC_PARAMS,
)


def _sc_scatter_body(y_hbm, src3_hbm, dst3_hbm, zeros_hbm, o0_hbm, o1_hbm,
                     sidx, didx, rows, acc, sem):
    cid = lax.axis_index("c")
    sid = lax.axis_index("s")
    gw = cid * NS + sid
    row0 = pl.multiple_of(sid * RPS, 8)

    pltpu.sync_copy(zeros_hbm.at[pl.ds(row0, RPS)],
                    acc.at[pl.ds(row0, RPS)])
    pltpu.async_copy(src3_hbm.at[gw], sidx, sem).wait()
    pltpu.async_copy(dst3_hbm.at[gw], didx, sem).wait()
    plsc.subcore_barrier()

    @pl.loop(0, NWIN)
    def _(j):
        pltpu.async_copy(y_hbm.at[sidx.at[j]], rows, sem).wait()
        pltpu.sync_copy(rows, acc.at[didx.at[j]], add=True)

    plsc.subcore_barrier()

    @pl.when(cid == 0)
    def _():
        pltpu.sync_copy(acc.at[pl.ds(row0, RPS)],
                        o0_hbm.at[pl.ds(row0, RPS)])

    @pl.when(cid == 1)
    def _():
        pltpu.sync_copy(acc.at[pl.ds(row0, RPS)],
                        o1_hbm.at[pl.ds(row0, RPS)])


_sc_scatter = pl.kernel(
    _sc_scatter_body,
    out_type=(jax.ShapeDtypeStruct((NPAD, D), F32),
              jax.ShapeDtypeStruct((NPAD, D), F32)),
    mesh=_MESH,
    scratch_types=[
        pltpu.VMEM((NWIN, WIN), jnp.int32),
        pltpu.VMEM((NWIN, WIN), jnp.int32),
        pltpu.VMEM((WIN, D), F32),
        pltpu.VMEM_SHARED((NPAD, D), F32),
        pltpu.SemaphoreType.DMA,
    ],
    compiler_params=_SC_PARAMS,
)


BLK = 1000
HIGH = lax.Precision.HIGHEST


def _tc_matmul_body(x_ref, w_ref, o_ref):
    o_ref[...] = jnp.dot(x_ref[...], w_ref[...], preferred_element_type=F32,
                         precision=HIGH)


_tc_matmul = pl.pallas_call(
    _tc_matmul_body,
    grid=(N // BLK,),
    in_specs=[pl.BlockSpec((BLK, D), lambda i: (i, 0)),
              pl.BlockSpec((D, H), lambda i: (0, 0))],
    out_specs=pl.BlockSpec((BLK, H), lambda i: (i, 0)),
    out_shape=jax.ShapeDtypeStruct((N, H), F32),
)


def _tc_scale_body(d0_ref, d1_ref, xw_ref, y_ref, dinv_ref):
    deg = d0_ref[...] + d1_ref[...] + 1.0
    dinv = lax.rsqrt(deg)
    dinv_ref[...] = dinv
    y_ref[...] = xw_ref[...] * dinv


_tc_scale = pl.pallas_call(
    _tc_scale_body,
    grid=(N // BLK,),
    in_specs=[pl.BlockSpec((BLK, 1), lambda i: (i, 0)),
              pl.BlockSpec((BLK, 1), lambda i: (i, 0)),
              pl.BlockSpec((BLK, H), lambda i: (i, 0))],
    out_specs=[pl.BlockSpec((BLK, H), lambda i: (i, 0)),
               pl.BlockSpec((BLK, 1), lambda i: (i, 0))],
    out_shape=[jax.ShapeDtypeStruct((N, H), F32),
               jax.ShapeDtypeStruct((N, 1), F32)],
)


def _tc_mid_body(a0_ref, a1_ref, y1_ref, dinv_ref, b1_ref, w2_ref, o_ref):
    dinv = dinv_ref[...]
    t = (a0_ref[...] + a1_ref[...] + y1_ref[...]) * dinv + b1_ref[...]
    h = jnp.maximum(t, 0.0)
    o_ref[...] = jnp.dot(h, w2_ref[...], preferred_element_type=F32,
                         precision=HIGH) * dinv


_tc_mid = pl.pallas_call(
    _tc_mid_body,
    grid=(N // BLK,),
    in_specs=[pl.BlockSpec((BLK, H), lambda i: (i, 0)),
              pl.BlockSpec((BLK, H), lambda i: (i, 0)),
              pl.BlockSpec((BLK, H), lambda i: (i, 0)),
              pl.BlockSpec((BLK, 1), lambda i: (i, 0)),
              pl.BlockSpec((1, H), lambda i: (0, 0)),
              pl.BlockSpec((H, H), lambda i: (0, 0))],
    out_specs=pl.BlockSpec((BLK, H), lambda i: (i, 0)),
    out_shape=jax.ShapeDtypeStruct((N, H), F32),
)


def _tc_head_body(c0_ref, c1_ref, y2_ref, dinv_ref, b2_ref, batch_ref, o_ref,
                  pooled):
    i = pl.program_id(0)

    @pl.when(i == 0)
    def _():
        pooled[...] = jnp.full((G, H), -jnp.inf, F32)

    blk = (c0_ref[...] + c1_ref[...] + y2_ref[...]) * dinv_ref[...] \
        + b2_ref[...]
    bb = batch_ref[...]
    glo = jnp.min(bb)
    ghi = jnp.max(bb)

    def body(g, carry):
        v = jnp.where(bb == g, blk, -jnp.inf)
        red = jnp.max(v, axis=0, keepdims=True)
        pooled[pl.ds(g, 1), :] = jnp.maximum(pooled[pl.ds(g, 1), :], red)
        return carry

    lax.fori_loop(glo, ghi + 1, body, 0)

    @pl.when(i == N // BLK - 1)
    def _():
        p = pooled[...]
        mx = jnp.max(p, axis=1, keepdims=True)
        s = jnp.sum(jnp.exp(p - mx), axis=1, keepdims=True)
        o_ref[...] = p - mx - jnp.log(s)


_tc_head = pl.pallas_call(
    _tc_head_body,
    grid=(N // BLK,),
    in_specs=[pl.BlockSpec((BLK, H), lambda i: (i, 0)),
              pl.BlockSpec((BLK, H), lambda i: (i, 0)),
              pl.BlockSpec((BLK, H), lambda i: (i, 0)),
              pl.BlockSpec((BLK, 1), lambda i: (i, 0)),
              pl.BlockSpec((1, H), lambda i: (0, 0)),
              pl.BlockSpec((BLK, 1), lambda i: (i, 0))],
    out_specs=pl.BlockSpec((G, H), lambda i: (0, 0)),
    out_shape=jax.ShapeDtypeStruct((G, H), F32),
    scratch_shapes=[pltpu.VMEM((G, H), F32)],
)


def kernel(x, edge_index, batch, W1, b1, W2, b2):
    src = edge_index[0].astype(jnp.int32)
    dst = edge_index[1].astype(jnp.int32)
    src3 = src.reshape(NW, NWIN, WIN)
    dst3 = dst.reshape(NW, NWIN, WIN)
    zeros = jnp.zeros((NPAD, D), F32)

    deg0, deg1 = _sc_degree(dst)
    xw1 = _tc_matmul(x, W1)
    y1, dinv = _tc_scale(deg0[:N].reshape(N, 1), deg1[:N].reshape(N, 1), xw1)
    a0, a1 = _sc_scatter(y1, src3, dst3, zeros)
    y2 = _tc_mid(a0[:N], a1[:N], y1, dinv, b1.reshape(1, H), W2)
    c0, c1 = _sc_scatter(y2, src3, dst3, zeros)
    return _tc_head(c0[:N], c1[:N], y2, dinv, b2.reshape(1, H),
                    batch.astype(jnp.int32).reshape(N, 1))
